# Initial kernel scaffold; baseline (speedup 1.0000x reference)
import jax
import jax.numpy as jnp
from jax.experimental import pallas as pl


def kernel(flat_ids, cu_seqlens):
    raise NotImplementedError("write your pallas kernel here")


# speedup vs baseline: 11.0163x; 11.0163x over previous
import functools

import jax
import jax.numpy as jnp
from jax import lax
from jax.experimental import pallas as pl
from jax.experimental.pallas import tpu as pltpu
from jax.experimental.pallas import tpu_sc as plsc

B = 16
MAX_SEQLEN = 4096
TOTAL = 32768
CLS_ID = 101
SEP_ID = 102
OUT_LEN = MAX_SEQLEN + 2
CHUNKS = (OUT_LEN + 15) // 16
CHUNKS_C = (CHUNKS + 1) // 2
HALF = CHUNKS_C * 16
WIN = HALF + 16
BASE_MAX = TOTAL - WIN
IN_V = 8 + (TOTAL + HALF - BASE_MAX) + HALF + 16


def _row_body(flat_hbm, cu_hbm, out_hbm, cu_v, in_v, out_v):
    c = lax.axis_index("c")
    s = lax.axis_index("s")

    pltpu.sync_copy(cu_hbm, cu_v.at[pl.ds(0, 17)])
    iota = lax.broadcasted_iota(jnp.int32, (16,), 0)
    start = cu_v[pl.ds(s, 16)][0]
    end = cu_v[pl.ds(s + 1, 16)][0]
    ln = end - start

    col0 = c * HALF
    a = start + col0
    base = jnp.clip(jnp.bitwise_and(a - 8, -8), 0, BASE_MAX)
    base = pl.multiple_of(base, 8)
    shift = a - base
    pltpu.sync_copy(flat_hbm.at[pl.ds(base, WIN)], in_v.at[pl.ds(8, WIN)])

    def chunk(i, carry):
        col = iota + col0 + i * 16
        vals = in_v[pl.ds(7 + shift + i * 16, 16)]
        vals = jnp.where(col <= ln, vals, 0)
        vals = jnp.where(col == 0, CLS_ID, vals)
        vals = jnp.where(col == ln + 1, SEP_ID, vals)
        out_v[pl.ds(i * 16, 16)] = vals
        return carry

    lax.fori_loop(0, CHUNKS_C, chunk, 0)

    @pl.when(c == 0)
    def _():
        pltpu.sync_copy(out_v.at[pl.ds(0, HALF)], out_hbm.at[s, pl.ds(0, HALF)])

    @pl.when(c == 1)
    def _():
        pltpu.sync_copy(
            out_v.at[pl.ds(0, OUT_LEN - HALF)],
            out_hbm.at[s, pl.ds(HALF, OUT_LEN - HALF)],
        )


@functools.partial(
    pl.kernel,
    out_type=jax.ShapeDtypeStruct((B, OUT_LEN), jnp.int32),
    mesh=plsc.VectorSubcoreMesh(core_axis_name="c", subcore_axis_name="s"),
    compiler_params=pltpu.CompilerParams(use_tc_tiling_on_sc=False),
    scratch_types=[
        pltpu.VMEM((32,), jnp.int32),
        pltpu.VMEM((IN_V,), jnp.int32),
        pltpu.VMEM((HALF,), jnp.int32),
    ],
)
def _sc_merge(flat_hbm, cu_hbm, out_hbm, cu_v, in_v, out_v):
    _row_body(flat_hbm, cu_hbm, out_hbm, cu_v, in_v, out_v)


def kernel(flat_ids, cu_seqlens):
    return _sc_merge(flat_ids, cu_seqlens.astype(jnp.int32))


# speedup vs baseline: 11.3116x; 1.0268x over previous
import functools

import jax
import jax.numpy as jnp
from jax import lax
from jax.experimental import pallas as pl
from jax.experimental.pallas import tpu as pltpu
from jax.experimental.pallas import tpu_sc as plsc

B = 16
MAX_SEQLEN = 4096
TOTAL = 32768
CLS_ID = 101
SEP_ID = 102
OUT_LEN = MAX_SEQLEN + 2
HALF = 2064
CHUNKS_LOOP = 132
WIN = HALF + 16
STAGE = 24
BASE_MAX = TOTAL - WIN
SHIFT_MAX = TOTAL + HALF - BASE_MAX
IN_V = STAGE + SHIFT_MAX + 16 * CHUNKS_LOOP + 16


def _row_body(flat_hbm, cu_hbm, out_hbm, cu_v, in_v, out_v):
    c = lax.axis_index("c")
    s = lax.axis_index("s")

    pltpu.sync_copy(cu_hbm, cu_v.at[pl.ds(0, 17)])
    iota = lax.broadcasted_iota(jnp.int32, (16,), 0)
    start = cu_v[pl.ds(s, 16)][0]
    end = cu_v[pl.ds(s + 1, 16)][0]
    ln = end - start

    col0 = c * HALF
    a = start + col0
    base = jnp.clip(jnp.bitwise_and(a - 8, -8), 0, BASE_MAX)
    base = pl.multiple_of(base, 8)
    shift = a - base
    pltpu.sync_copy(flat_hbm.at[pl.ds(base, WIN)], in_v.at[pl.ds(STAGE, WIN)])

    @pl.when(c == 0)
    def _():
        in_v[pl.ds(STAGE - 16 + shift, 16)] = jnp.where(iota == 15, CLS_ID, 0)
    sep_at = STAGE - 1 + shift + (ln + 1 - col0)
    in_sep = (ln + 1 >= col0) & (ln + 1 < col0 + 16 * CHUNKS_LOOP)
    sep_off = jnp.where(in_sep, sep_at, 0)

    @pl.when(in_sep)
    def _():
        old = in_v[pl.ds(sep_off, 16)]
        in_v[pl.ds(sep_off, 16)] = jnp.where(iota == 0, SEP_ID, old)

    lim = ln + 1 - col0

    @plsc.parallel_loop(0, 16 * CHUNKS_LOOP, step=16, unroll=4)
    def _(i):
        vals = in_v[pl.ds(STAGE - 1 + shift + i, 16)]
        keep = iota + i <= lim
        out_v[pl.ds(i, 16)] = jnp.where(keep, vals, 0)

    @pl.when(c == 0)
    def _():
        pltpu.sync_copy(out_v.at[pl.ds(0, HALF)], out_hbm.at[s, pl.ds(0, HALF)])

    @pl.when(c == 1)
    def _():
        pltpu.sync_copy(
            out_v.at[pl.ds(0, OUT_LEN - HALF)],
            out_hbm.at[s, pl.ds(HALF, OUT_LEN - HALF)],
        )


@functools.partial(
    pl.kernel,
    out_type=jax.ShapeDtypeStruct((B, OUT_LEN), jnp.int32),
    mesh=plsc.VectorSubcoreMesh(core_axis_name="c", subcore_axis_name="s"),
    compiler_params=pltpu.CompilerParams(use_tc_tiling_on_sc=False),
    scratch_types=[
        pltpu.VMEM((32,), jnp.int32),
        pltpu.VMEM((IN_V,), jnp.int32),
        pltpu.VMEM((16 * CHUNKS_LOOP,), jnp.int32),
    ],
)
def _sc_merge(flat_hbm, cu_hbm, out_hbm, cu_v, in_v, out_v):
    _row_body(flat_hbm, cu_hbm, out_hbm, cu_v, in_v, out_v)


def kernel(flat_ids, cu_seqlens):
    return _sc_merge(flat_ids, cu_seqlens.astype(jnp.int32))


# speedup vs baseline: 11.8443x; 1.0471x over previous
import functools

import jax
import jax.numpy as jnp
from jax import lax
from jax.experimental import pallas as pl
from jax.experimental.pallas import tpu as pltpu
from jax.experimental.pallas import tpu_sc as plsc

B = 16
MAX_SEQLEN = 4096
TOTAL = 32768
CLS_ID = 101
SEP_ID = 102
OUT_LEN = MAX_SEQLEN + 2
HALF = 2048
NPIECE = HALF // 128
CHUNKS_LOOP = HALF // 16
WIN = HALF + 16
STAGE = 24
BASE_MAX = TOTAL - WIN
SHIFT_MAX = TOTAL + HALF - BASE_MAX
IN_V = STAGE + SHIFT_MAX + HALF + 16


def _row_body(flat_hbm, cu_hbm, out_hbm, cu_v, in_v, out_v, sem):
    c = lax.axis_index("c")
    s = lax.axis_index("s")

    pltpu.sync_copy(cu_hbm, cu_v.at[pl.ds(0, 17)])
    iota = lax.broadcasted_iota(jnp.int32, (16,), 0)
    start = cu_v[pl.ds(s, 16)][0]
    end = cu_v[pl.ds(s + 1, 16)][0]
    ln = end - start

    col0 = c * HALF
    a = start + col0
    base = jnp.clip(jnp.bitwise_and(a - 8, -8), 0, BASE_MAX)
    base = pl.multiple_of(base, 8)
    shift = a - base
    pltpu.sync_copy(flat_hbm.at[pl.ds(base, WIN)], in_v.at[pl.ds(STAGE, WIN)])

    @pl.when(c == 0)
    def _():
        in_v[pl.ds(STAGE - 16 + shift, 16)] = jnp.where(iota == 15, CLS_ID, 0)
    sep_at = STAGE - 1 + shift + (ln + 1 - col0)
    in_sep = (ln + 1 >= col0) & (ln + 1 < col0 + HALF)
    sep_off = jnp.where(in_sep, sep_at, 0)

    @pl.when(in_sep)
    def _():
        old = in_v[pl.ds(sep_off, 16)]
        in_v[pl.ds(sep_off, 16)] = jnp.where(iota == 0, SEP_ID, old)

    lim = ln + 1 - col0

    @plsc.parallel_loop(0, HALF, step=16, unroll=4)
    def _(i):
        vals = in_v[pl.ds(STAGE - 1 + shift + i, 16)]
        keep = iota + i <= lim
        out_v[pl.ds(i, 16)] = jnp.where(keep, vals, 0)

    copies = [
        pltpu.async_copy(
            out_v.at[pl.ds(t * 128, 128)],
            out_hbm.at[s, pl.ds(col0 + t * 128, 128)],
            sem,
        )
        for t in range(NPIECE)
    ]
    for cp in copies:
        cp.wait()


@functools.partial(
    pl.kernel,
    out_type=jax.ShapeDtypeStruct((B, OUT_LEN), jnp.int32),
    mesh=plsc.VectorSubcoreMesh(core_axis_name="c", subcore_axis_name="s"),
    compiler_params=pltpu.CompilerParams(use_tc_tiling_on_sc=True),
    scratch_types=[
        pltpu.VMEM((32,), jnp.int32),
        pltpu.VMEM((IN_V,), jnp.int32),
        pltpu.VMEM((HALF,), jnp.int32),
        pltpu.SemaphoreType.DMA,
    ],
)
def _sc_merge(flat_hbm, cu_hbm, out_hbm, cu_v, in_v, out_v, sem):
    _row_body(flat_hbm, cu_hbm, out_hbm, cu_v, in_v, out_v, sem)


def kernel(flat_ids, cu_seqlens):
    cu = cu_seqlens.astype(jnp.int32)
    out = _sc_merge(flat_ids, cu)
    ln = cu[1:] - cu[:-1]
    last_tok = flat_ids[jnp.clip(cu[1:] - 1, 0, TOTAL - 1)]
    c0 = jnp.where(ln == MAX_SEQLEN, last_tok,
                   jnp.where(ln == MAX_SEQLEN - 1, SEP_ID, 0))
    c1 = jnp.where(ln == MAX_SEQLEN, SEP_ID, 0)
    tail = jnp.stack([c0, c1], axis=1).astype(out.dtype)
    return lax.dynamic_update_slice(out, tail, (0, MAX_SEQLEN))
